# baseline (speedup 1.0000x reference)
import functools

import jax
import jax.numpy as jnp
from jax import lax
from jax.experimental import pallas as pl
from jax.experimental.pallas import tpu as pltpu
from jax.experimental.pallas import tpu_sc as plsc

B = 16384
F = 26
V = 100000
D = 64
R = F * D


def _build_sc_gather():
    info = plsc.get_sparse_core_info()
    NC, NS, L = info.num_cores, info.num_subcores, info.num_lanes
    NW = NC * NS
    rows_per_w = R // NW
    QB = B // 4
    G = QB // L
    mesh = plsc.VectorSubcoreMesh(core_axis_name="c", subcore_axis_name="s")

    @functools.partial(
        pl.kernel,
        mesh=mesh,
        compiler_params=pltpu.CompilerParams(
            use_tc_tiling_on_sc=True, needs_layout_passes=False),
        out_type=jax.ShapeDtypeStruct((R, B), jnp.float32),
        scratch_types=[
            pltpu.VMEM((V,), jnp.float32),
            pltpu.VMEM((B,), jnp.int32),
            pltpu.VMEM((QB,), jnp.float32),
            pltpu.VMEM((QB,), jnp.float32),
            pltpu.SemaphoreType.DMA,
            pltpu.SemaphoreType.DMA,
        ],
    )
    def sc_gather(xsT_hbm, tab_hbm, out_hbm, slab_v, idx_v, ob0, ob1, sm0, sm1):
        wid = lax.axis_index("s") * NC + lax.axis_index("c")
        row0 = wid * rows_per_w
        obufs = (ob0, ob1)
        sems = (sm0, sm1)

        def out_copy(r, q, p):
            return pltpu.make_async_copy(
                obufs[p], out_hbm.at[r, pl.ds(q * QB, QB)], sems[p])

        def row_body(r, carry):
            f = r // D

            @pl.when(jnp.logical_or(r == row0, r % D == 0))
            def _load_idx():
                pltpu.sync_copy(xsT_hbm.at[f], idx_v)

            pltpu.sync_copy(tab_hbm.at[r], slab_v)
            for q in range(4):
                p = q % 2
                if q >= 2:
                    out_copy(r, q - 2, p).wait()
                else:

                    @pl.when(r > row0)
                    def _drain():
                        out_copy(r - 1, q + 2, p).wait()

                base = q * QB
                ob = obufs[p]

                @plsc.parallel_loop(0, G, unroll=8)
                def _gather(i):
                    ob[pl.ds(i * L, L)] = plsc.load_gather(
                        slab_v, [idx_v[pl.ds(base + i * L, L)]])

                out_copy(r, q, p).start()
            return carry

        lax.fori_loop(row0, row0 + rows_per_w, row_body, 0)
        rlast = row0 + rows_per_w - 1
        for q in range(2, 4):
            out_copy(rlast, q, q % 2).wait()

    return sc_gather


_sc_gather = _build_sc_gather()


def kernel(x_sparse, x_varlen, x_dense, tables):
    xsT = x_sparse.T
    tabT = tables.transpose(0, 2, 1).reshape(R, V)
    out_T = _sc_gather(xsT, tabT)
    return out_T.reshape(F, D, B).transpose(2, 0, 1)
